# baseline (speedup 1.0000x reference)
import jax
import jax.numpy as jnp
from jax.experimental import pallas as pl
from jax.experimental.pallas import tpu as pltpu

N = 10000
D = 128
FH1 = 64
FH2 = 32
GH1 = 32
GH2 = 16
LAT = FH2 + GH2
EPS = 1e-3

BMC = 200
NCH = N // BMC
NBUF = 2
NPIN = 2
TOT = 2 * NCH
LAST = TOT - NPIN


def _bn(x, g, b, rm, rv):
    return (x - rm) / jnp.sqrt(rv + EPS) * g + b


def _elu(x):
    return jnp.where(x > 0, x, jnp.exp(x) - 1.0)


def _fused_kernel(x_ref, adj_hbm,
                  W1_ref, b1_ref, g1_ref, be1_ref, rm1_ref, rv1_ref,
                  W2_ref, b2_ref, g2_ref, be2_ref, rm2_ref, rv2_ref,
                  Wg1_ref, Wg2_ref, Wg3_ref,
                  Wd1_ref, bd1_ref, gd1_ref, bed1_ref, rmd1_ref, rvd1_ref,
                  Wd2_ref, bd2_ref, gd2_ref, bed2_ref, rmd2_ref, rvd2_ref,
                  big_ref, dec_ref,
                  bufs, pins, s1_sc, s2_sc, sems):

    def start_copy(step, slot):
        c = jax.lax.rem(step, NCH)
        src = adj_hbm.at[pl.ds(c * BMC, BMC), :]

        @pl.when(step == NCH - 2)
        def _to_pin0():
            pltpu.make_async_copy(src, pins.at[0], sems.at[slot]).start()

        @pl.when(step == NCH - 1)
        def _to_pin1():
            pltpu.make_async_copy(src, pins.at[1], sems.at[slot]).start()

        @pl.when((step != NCH - 2) & (step != NCH - 1))
        def _to_slot():
            pltpu.make_async_copy(src, bufs.at[slot], sems.at[slot]).start()

    def wait_copy(slot):
        pltpu.make_async_copy(adj_hbm.at[pl.ds(0, BMC), :],
                              bufs.at[slot], sems.at[slot]).wait()

    for k in range(NBUF):
        pltpu.make_async_copy(adj_hbm.at[pl.ds(k * BMC, BMC), :],
                              bufs.at[k], sems.at[k]).start()

    h = jnp.dot(x_ref[...], W1_ref[...], preferred_element_type=jnp.float32)
    h = _elu(_bn(h + b1_ref[...], g1_ref[...], be1_ref[...],
                 rm1_ref[...], rv1_ref[...]))
    f = jnp.dot(h, W2_ref[...], preferred_element_type=jnp.float32)
    f = _elu(_bn(f + b2_ref[...], g2_ref[...], be2_ref[...],
                 rm2_ref[...], rv2_ref[...]))
    big_ref[:, :FH2] = f
    s1_sc[...] = jnp.dot(f, Wg1_ref[...], preferred_element_type=jnp.float32)

    def pass1_compute(a_ref, rows):
        h1 = jnp.dot(a_ref[...], s1_sc[...], preferred_element_type=jnp.float32)
        s2_sc[rows, :] = jnp.concatenate(
            [jnp.dot(h1, Wg2_ref[...], preferred_element_type=jnp.float32),
             jnp.dot(h1, Wg3_ref[...], preferred_element_type=jnp.float32)],
            axis=1)

    def pass2_compute(a_ref, rows):
        out2 = jnp.dot(a_ref[...], s2_sc[...], preferred_element_type=jnp.float32)
        mu = out2[:, :GH2]
        big_ref[rows, FH2:FH2 + GH2] = mu
        big_ref[rows, FH2 + GH2:] = out2[:, GH2:]
        z = jnp.concatenate([big_ref[rows, :FH2], mu], axis=1)
        d = jnp.dot(z, Wd1_ref[...], preferred_element_type=jnp.float32)
        d = _elu(_bn(d + bd1_ref[...], gd1_ref[...], bed1_ref[...],
                     rmd1_ref[...], rvd1_ref[...]))
        dec = jnp.dot(d, Wd2_ref[...], preferred_element_type=jnp.float32)
        dec_ref[rows, :] = jax.nn.relu(
            _bn(dec + bd2_ref[...], gd2_ref[...], bed2_ref[...],
                rmd2_ref[...], rvd2_ref[...]))

    def process(s, slot):
        c = jax.lax.rem(s, NCH)
        p = s // NCH
        rows = pl.ds(c * BMC, BMC)
        wait_copy(slot)

        @pl.when((p == 0) & (s < NCH - 2))
        def _pass1():
            pass1_compute(bufs.at[slot], rows)

        @pl.when(s == NCH - 2)
        def _pass1_pin0():
            pass1_compute(pins.at[0], rows)

        @pl.when(s == NCH - 1)
        def _pass1_pin1():
            pass1_compute(pins.at[1], rows)

        @pl.when(p == 1)
        def _pass2():
            pass2_compute(bufs.at[slot], rows)

        @pl.when(s + NBUF < LAST)
        def _next():
            start_copy(s + NBUF, slot)

    def body(j, _):
        base = j * NBUF
        for k in range(NBUF):
            process(base + k, k)
        return 0

    jax.lax.fori_loop(0, LAST // NBUF, body, 0)

    pass2_compute(pins.at[0], pl.ds((NCH - 2) * BMC, BMC))
    pass2_compute(pins.at[1], pl.ds((NCH - 1) * BMC, BMC))


def _row(v):
    return v.reshape(1, -1)


def kernel(x, adj, W1, b1, g1, be1, rm1, rv1, W2, b2, g2, be2, rm2, rv2,
           Wg1, Wg2, Wg3,
           Wd1, bd1, gd1, bed1, rmd1, rvd1,
           Wd2, bd2, gd2, bed2, rmd2, rvd2):
    f32 = jnp.float32
    vmem = pl.BlockSpec(memory_space=pltpu.VMEM)

    big, decoded_x = pl.pallas_call(
        _fused_kernel,
        in_specs=[vmem, pl.BlockSpec(memory_space=pl.ANY)] + [vmem] * 27,
        out_specs=[vmem] * 2,
        out_shape=[jax.ShapeDtypeStruct((N, FH2 + 2 * GH2), f32),
                   jax.ShapeDtypeStruct((N, D), f32)],
        scratch_shapes=[pltpu.VMEM((NBUF, BMC, N), f32),
                        pltpu.VMEM((2, BMC, N), f32),
                        pltpu.VMEM((N, GH1), f32),
                        pltpu.VMEM((N, 2 * GH2), f32),
                        pltpu.SemaphoreType.DMA((NBUF,))],
        compiler_params=pltpu.CompilerParams(
            vmem_limit_bytes=64 * 1024 * 1024),
    )(x, adj,
      W1, _row(b1), _row(g1), _row(be1), _row(rm1), _row(rv1),
      W2, _row(b2), _row(g2), _row(be2), _row(rm2), _row(rv2),
      Wg1, Wg2, Wg3,
      Wd1, _row(bd1), _row(gd1), _row(bed1), _row(rmd1), _row(rvd1),
      Wd2, _row(bd2), _row(gd2), _row(bed2), _row(rmd2), _row(rvd2))

    feat_x = big[:, :FH2]
    gcn_mu = big[:, FH2:FH2 + GH2]
    gcn_logstd = big[:, FH2 + GH2:]
    z = big[:, :LAT]
    return (gcn_mu, gcn_logstd, feat_x, gcn_mu, z, decoded_x)


# speedup vs baseline: 1.0770x; 1.0770x over previous
import jax
import jax.numpy as jnp
from jax.experimental import pallas as pl
from jax.experimental.pallas import tpu as pltpu

N = 10000
D = 128
FH1 = 64
FH2 = 32
GH1 = 32
GH2 = 16
LAT = FH2 + GH2
EPS = 1e-3

BMC = 200
NCH = N // BMC
NBUF = 3
TOT = 2 * NCH
LAST = TOT - 1


def _bn(x, g, b, rm, rv):
    return (x - rm) / jnp.sqrt(rv + EPS) * g + b


def _elu(x):
    return jnp.where(x > 0, x, jnp.exp(x) - 1.0)


def _fused_kernel(x_ref, adj_hbm,
                  W1_ref, b1_ref, g1_ref, be1_ref, rm1_ref, rv1_ref,
                  W2_ref, b2_ref, g2_ref, be2_ref, rm2_ref, rv2_ref,
                  Wg1_ref, Wg2_ref, Wg3_ref,
                  Wd1_ref, bd1_ref, gd1_ref, bed1_ref, rmd1_ref, rvd1_ref,
                  Wd2_ref, bd2_ref, gd2_ref, bed2_ref, rmd2_ref, rvd2_ref,
                  big_ref, dec_ref,
                  bufs, pin_sc, s1_sc, s2_sc, sems):

    def start_copy(step, slot):
        c = jax.lax.rem(step, NCH)
        src = adj_hbm.at[pl.ds(c * BMC, BMC), :]

        @pl.when(step == NCH - 1)
        def _to_pin():
            pltpu.make_async_copy(src, pin_sc, sems.at[slot]).start()

        @pl.when(step != NCH - 1)
        def _to_slot():
            pltpu.make_async_copy(src, bufs.at[slot], sems.at[slot]).start()

    def wait_copy(slot):
        pltpu.make_async_copy(adj_hbm.at[pl.ds(0, BMC), :],
                              bufs.at[slot], sems.at[slot]).wait()

    for k in range(NBUF):
        pltpu.make_async_copy(adj_hbm.at[pl.ds(k * BMC, BMC), :],
                              bufs.at[k], sems.at[k]).start()

    h = jnp.dot(x_ref[...], W1_ref[...], preferred_element_type=jnp.float32)
    h = _elu(_bn(h + b1_ref[...], g1_ref[...], be1_ref[...],
                 rm1_ref[...], rv1_ref[...]))
    f = jnp.dot(h, W2_ref[...], preferred_element_type=jnp.float32)
    f = _elu(_bn(f + b2_ref[...], g2_ref[...], be2_ref[...],
                 rm2_ref[...], rv2_ref[...]))
    big_ref[:, :FH2] = f
    s1_sc[...] = jnp.dot(f, Wg1_ref[...], preferred_element_type=jnp.float32)

    def pass1_compute(a_ref, rows):
        h1 = jnp.dot(a_ref[...], s1_sc[...], preferred_element_type=jnp.float32)
        s2_sc[rows, :] = jnp.concatenate(
            [jnp.dot(h1, Wg2_ref[...], preferred_element_type=jnp.float32),
             jnp.dot(h1, Wg3_ref[...], preferred_element_type=jnp.float32)],
            axis=1)

    def pass2_compute(a_ref, rows):
        out2 = jnp.dot(a_ref[...], s2_sc[...], preferred_element_type=jnp.float32)
        mu = out2[:, :GH2]
        big_ref[rows, FH2:FH2 + GH2] = mu
        big_ref[rows, FH2 + GH2:] = out2[:, GH2:]
        z = jnp.concatenate([big_ref[rows, :FH2], mu], axis=1)
        d = jnp.dot(z, Wd1_ref[...], preferred_element_type=jnp.float32)
        d = _elu(_bn(d + bd1_ref[...], gd1_ref[...], bed1_ref[...],
                     rmd1_ref[...], rvd1_ref[...]))
        dec = jnp.dot(d, Wd2_ref[...], preferred_element_type=jnp.float32)
        dec_ref[rows, :] = jax.nn.relu(
            _bn(dec + bd2_ref[...], gd2_ref[...], bed2_ref[...],
                rmd2_ref[...], rvd2_ref[...]))

    def process(s, slot):
        c = jax.lax.rem(s, NCH)
        p = s // NCH
        rows = pl.ds(c * BMC, BMC)
        wait_copy(slot)

        @pl.when((p == 0) & (s != NCH - 1))
        def _pass1():
            pass1_compute(bufs.at[slot], rows)

        @pl.when(s == NCH - 1)
        def _pass1_pin():
            pass1_compute(pin_sc, rows)

        @pl.when(p == 1)
        def _pass2():
            pass2_compute(bufs.at[slot], rows)

        @pl.when(s + NBUF < LAST)
        def _next():
            start_copy(s + NBUF, slot)

    def body(j, _):
        base = j * NBUF
        for k in range(NBUF):
            process(base + k, k)
        return 0

    jax.lax.fori_loop(0, LAST // NBUF, body, 0)

    pass2_compute(pin_sc, pl.ds((NCH - 1) * BMC, BMC))


def _row(v):
    return v.reshape(1, -1)


def kernel(x, adj, W1, b1, g1, be1, rm1, rv1, W2, b2, g2, be2, rm2, rv2,
           Wg1, Wg2, Wg3,
           Wd1, bd1, gd1, bed1, rmd1, rvd1,
           Wd2, bd2, gd2, bed2, rmd2, rvd2):
    f32 = jnp.float32
    vmem = pl.BlockSpec(memory_space=pltpu.VMEM)

    big, decoded_x = pl.pallas_call(
        _fused_kernel,
        in_specs=[vmem, pl.BlockSpec(memory_space=pl.ANY)] + [vmem] * 27,
        out_specs=[vmem] * 2,
        out_shape=[jax.ShapeDtypeStruct((N, FH2 + 2 * GH2), f32),
                   jax.ShapeDtypeStruct((N, D), f32)],
        scratch_shapes=[pltpu.VMEM((NBUF, BMC, N), f32),
                        pltpu.VMEM((BMC, N), f32),
                        pltpu.VMEM((N, GH1), f32),
                        pltpu.VMEM((N, 2 * GH2), f32),
                        pltpu.SemaphoreType.DMA((NBUF,))],
        compiler_params=pltpu.CompilerParams(
            vmem_limit_bytes=64 * 1024 * 1024),
    )(x, adj,
      W1, _row(b1), _row(g1), _row(be1), _row(rm1), _row(rv1),
      W2, _row(b2), _row(g2), _row(be2), _row(rm2), _row(rv2),
      Wg1, Wg2, Wg3,
      Wd1, _row(bd1), _row(gd1), _row(bed1), _row(rmd1), _row(rvd1),
      Wd2, _row(bd2), _row(gd2), _row(bed2), _row(rmd2), _row(rvd2))

    feat_x = big[:, :FH2]
    gcn_mu = big[:, FH2:FH2 + GH2]
    gcn_logstd = big[:, FH2 + GH2:]
    z = big[:, :LAT]
    return (gcn_mu, gcn_logstd, feat_x, gcn_mu, z, decoded_x)
